# Initial kernel scaffold; baseline (speedup 1.0000x reference)
import jax
import jax.numpy as jnp
from jax.experimental import pallas as pl


def kernel(coordinates, W1_0, b1_0, W2_0, b2_0, Wskip0, W1_1, b1_1, W2_1, b2_1, Wclf, bclf):
    raise NotImplementedError("write your pallas kernel here")


# speedup vs baseline: 1.5707x; 1.5707x over previous
import numpy as np
import jax
import jax.numpy as jnp
from jax.experimental import pallas as pl
from jax.experimental.pallas import tpu as pltpu

_B, _N, _K = 4, 1024, 16
_AZ, _EL, _RAD, _HB = 4, 2, 2, 8
_NR, _NA = 3, 6
_TR = 0.3
_F0 = _AZ * _EL * _RAD * _HB
_T = 96
_NCLS = 40
_PF = 128
_NT = _NR * _NA
_BIG = np.float32(3.0e38)


def _f32(x):
    return x.astype(jnp.float32)


def _topk_kernel(d_ref, nidx_ref):
    key = d_ref[0]
    iota = jax.lax.broadcasted_iota(jnp.int32, (_N, _N), 1)
    for i in range(_K + 1):
        m = jnp.min(key, axis=1, keepdims=True)
        sel = jnp.min(jnp.where(key == m, iota, _N), axis=1, keepdims=True)
        if i > 0:
            nidx_ref[0, :, i - 1:i] = sel
        key = jnp.where(iota == sel, _BIG, key)


def _hist_kernel(bins_ref, gidx_ref, desc_ref):
    del gidx_ref
    bins = bins_ref[0]
    iota_f = jax.lax.broadcasted_iota(jnp.int32, (1, _F0), 1)
    acc = jnp.zeros((_N, _F0), jnp.float32)
    for k in range(_K):
        acc += _f32(bins[:, k:k + 1] == iota_f)
    desc_ref[0] = acc * np.float32(1.0 / _K)


_BR = 128


def _conv_core(sig, nidx, i3m, bcwm, w, b, ng_ref):
    iota_n = jax.lax.broadcasted_iota(jnp.int32, (_BR, _N), 1)
    for k in range(_K):
        nk = nidx[:, k:k + 1]
        oh = _f32(iota_n == nk)
        ng_ref[:, k * _PF:(k + 1) * _PF] = jax.lax.dot_general(
            oh, sig, (((1,), (0,)), ((), ())),
            preferred_element_type=jnp.float32,
            precision=jax.lax.Precision.HIGHEST)
    z = b * jnp.ones((_BR, 1), jnp.float32)
    for t in range(_NT):
        terms = []
        for m in range(3):
            sel = i3m[:, t * 3 + m:t * 3 + m + 1]
            wgt = bcwm[:, t * 3 + m:t * 3 + m + 1]
            gm = jnp.zeros((_BR, _PF), jnp.float32)
            for k in range(_K):
                gm = gm + _f32(sel == k) * ng_ref[:, k * _PF:(k + 1) * _PF]
            terms.append(wgt * gm)
        pt = (terms[0] + terms[1]) + terms[2]
        z = z + jax.lax.dot_general(
            pt, w[t * _PF:(t + 1) * _PF, :], (((1,), (0,)), ((), ())),
            preferred_element_type=jnp.float32)
    h = z[:, 0:_PF]
    for rot in range(1, _NA):
        h = jnp.maximum(h, z[:, rot * _PF:(rot + 1) * _PF])
    return h


def _conv_first_kernel(sig_ref, nidx_ref, i3_ref, bcw_ref, w_ref, b_ref,
                       out_ref, ng_ref):
    h = _conv_core(sig_ref[0], nidx_ref[0], i3_ref[0], bcw_ref[0],
                   w_ref[...], b_ref[...], ng_ref)
    out_ref[0] = jnp.maximum(h, 0.0)


def _conv_skip_kernel(sig_ref, nidx_ref, i3_ref, bcw_ref, w_ref, b_ref,
                      skip_ref, wskip_ref, out_ref, ng_ref):
    h = _conv_core(sig_ref[0], nidx_ref[0], i3_ref[0], bcw_ref[0],
                   w_ref[...], b_ref[...], ng_ref)
    sk = jax.lax.dot_general(skip_ref[0], wskip_ref[...],
                             (((1,), (0,)), ((), ())),
                             preferred_element_type=jnp.float32,
                             precision=jax.lax.Precision.HIGHEST)
    out_ref[0] = jnp.maximum(h + sk, 0.0)


def _head_kernel(sig_ref, wp_ref, bc_ref, out_ref):
    s = sig_ref[0]
    mean = jnp.sum(s, axis=0, keepdims=True) * np.float32(1.0 / _N)
    c = s - mean
    cov = jax.lax.dot_general(c, c, (((0,), (0,)), ((), ())),
                              preferred_element_type=jnp.float32,
                             precision=jax.lax.Precision.HIGHEST)
    cov = cov * np.float32(1.0 / (_N - 1))
    w3 = wp_ref[...]
    out = jnp.sum(cov[:, :, None] * w3, axis=(0, 1))
    out_ref[0] = out[None, :] + bc_ref[...]


def _per_batch_spec(shape):
    return pl.BlockSpec((1,) + shape, lambda b: (b,) + (0,) * len(shape))


def _shared_spec(shape):
    return pl.BlockSpec(shape, lambda b: (0,) * len(shape))


def _pack_conv_w(w, fin):
    wf = jnp.zeros((_T, _NR, _NA, _PF), jnp.float32).at[..., :fin].set(w)
    cols = []
    for rot in range(_NA):
        wr = jnp.roll(wf, -rot, axis=2)
        wr = wr.transpose(1, 2, 3, 0).reshape(_NR * _NA * _PF, _T)
        wr = jnp.zeros((_NR * _NA * _PF, _PF), jnp.float32
                       ).at[:, :_T].set(wr)
        cols.append(wr)
    return jnp.concatenate(cols, axis=1)


def _pack_bias(b):
    bp = jnp.zeros((_PF,), jnp.float32).at[:_T].set(b)
    return jnp.tile(bp, _NA)[None, :]


def _bary_weights(nidx, lx, ly):
    b, n, k = nidx.shape
    radii = _TR * (jnp.arange(1, _NR + 1, dtype=jnp.float32) / _NR)
    angles = 2.0 * np.pi * jnp.arange(_NA, dtype=jnp.float32) / _NA
    temp = jnp.stack([radii[:, None] * jnp.cos(angles)[None, :],
                      radii[:, None] * jnp.sin(angles)[None, :]], -1)
    planar = jnp.stack([lx, ly], -1)
    diff = planar[:, :, None, None, :, :] - temp[None, None, :, :, None, :]
    d2 = jnp.sum(diff * diff, -1)
    _, i3 = jax.lax.top_k(-d2, 3)
    planar_b = jnp.broadcast_to(planar[:, :, None, None, :, :],
                                (b, n, _NR, _NA, k, 2))
    tri = jnp.take_along_axis(planar_b, i3[..., None], axis=4)
    p1, p2, p3 = tri[..., 0, :], tri[..., 1, :], tri[..., 2, :]
    t = jnp.broadcast_to(temp[None, None, :, :, :], p1.shape)
    v0, v1, v2 = p2 - p1, p3 - p1, t - p1
    d00 = jnp.sum(v0 * v0, -1); d01 = jnp.sum(v0 * v1, -1)
    d11 = jnp.sum(v1 * v1, -1)
    d20 = jnp.sum(v2 * v0, -1); d21 = jnp.sum(v2 * v1, -1)
    denom = d00 * d11 - d01 * d01
    safe = jnp.where(jnp.abs(denom) < 1e-10, 1.0, denom)
    v = (d11 * d20 - d01 * d21) / safe
    w = (d00 * d21 - d01 * d20) / safe
    u = 1.0 - v - w
    bcw = jnp.stack([u, v, w], -1)
    bcw = jnp.where(jnp.abs(denom)[..., None] < 1e-10,
                    jnp.full_like(bcw, 1.0 / 3.0), bcw)
    nidx_b = jnp.broadcast_to(nidx[:, :, None, None, :], (b, n, _NR, _NA, k))
    gidx = jnp.take_along_axis(nidx_b, i3, axis=4)
    i3, bcw, gidx = jax.lax.optimization_barrier((i3, bcw, gidx))
    return i3, bcw, gidx


def _shot_bins(coords, nidx, normal, lx, ly, lz):
    b, n, k = nidx.shape
    r = jnp.sqrt(lx ** 2 + ly ** 2 + lz ** 2) + 1e-12
    max_r = jnp.max(r, axis=-1, keepdims=True)
    az = jnp.arctan2(ly, lx)
    az_bin = jnp.clip(jnp.floor((az + np.pi) / (2.0 * np.pi) * _AZ),
                      0, _AZ - 1).astype(jnp.int32)
    el_bin = (lz > 0).astype(jnp.int32)
    rad_bin = (r > max_r * 0.5).astype(jnp.int32)
    bi = jnp.arange(b)[:, None, None]
    nn = normal[bi, nidx]
    cosang = jnp.sum(nn * normal[:, :, None, :], axis=-1)
    h_bin = jnp.clip(jnp.floor((cosang + 1.0) * 0.5 * _HB),
                     0, _HB - 1).astype(jnp.int32)
    bins = ((az_bin * _EL + el_bin) * _RAD + rad_bin) * _HB + h_bin
    return bins.astype(jnp.int32)


def _loc_frames(coords, nidx):
    b, n, _ = coords.shape
    bi = jnp.arange(b)[:, None, None]
    neigh = coords[bi, nidx]
    centered = neigh - coords[:, :, None, :]
    cov = jnp.einsum('bnki,bnkj->bnij', centered, centered) / _K
    _, evecs = jnp.linalg.eigh(cov)
    normal = evecs[..., 0]
    x_ax = evecs[..., 2]
    sx = jnp.sign(jnp.sum(jnp.einsum('bnki,bni->bnk', centered, x_ax),
                          axis=-1) + 1e-9)
    x_ax = x_ax * sx[..., None]
    sz = jnp.sign(jnp.sum(jnp.einsum('bnki,bni->bnk', centered, normal),
                          axis=-1) + 1e-9)
    normal = normal * sz[..., None]
    y_ax = jnp.cross(normal, x_ax)
    lx = jnp.einsum('bnki,bni->bnk', centered, x_ax)
    ly = jnp.einsum('bnki,bni->bnk', centered, y_ax)
    lz = jnp.einsum('bnki,bni->bnk', centered, normal)
    return normal, lx, ly, lz


def kernel(coordinates, W1_0, b1_0, W2_0, b2_0, Wskip0, W1_1, b1_1, W2_1,
           b2_1, Wclf, bclf):
    coords = coordinates.astype(jnp.float32)
    sq = jnp.sum(coords * coords, axis=-1)
    d = (sq[:, :, None] + sq[:, None, :]
         - 2.0 * jnp.einsum('bnd,bmd->bnm', coords, coords))

    grid = (_B,)
    nidx = pl.pallas_call(
        _topk_kernel,
        grid=grid,
        in_specs=[_per_batch_spec((_N, _N))],
        out_specs=_per_batch_spec((_N, _K)),
        out_shape=jax.ShapeDtypeStruct((_B, _N, _K), jnp.int32),
    )(d)

    normal, lx, ly, lz = _loc_frames(coords, nidx)
    i3, bcw, gidx = _bary_weights(nidx, lx, ly)
    i3f = i3.astype(jnp.int32).reshape(_B, _N, _NT * 3)
    bcwf = bcw.reshape(_B, _N, _NT * 3)
    gidxf = gidx.astype(jnp.int32).reshape(_B, _N, _NT * 3)
    bins = _shot_bins(coords, nidx, normal, lx, ly, lz)

    desc = pl.pallas_call(
        _hist_kernel,
        grid=grid,
        in_specs=[_per_batch_spec((_N, _K)),
                  _per_batch_spec((_N, _NT * 3))],
        out_specs=_per_batch_spec((_N, _F0)),
        out_shape=jax.ShapeDtypeStruct((_B, _N, _F0), jnp.float32),
    )(bins, gidxf)

    kdim = _NR * _NA * _PF
    w10 = _pack_conv_w(W1_0, _F0)
    w20 = _pack_conv_w(W2_0, _T)
    w11 = _pack_conv_w(W1_1, _T)
    w21 = _pack_conv_w(W2_1, _T)
    bb10, bb20 = _pack_bias(b1_0), _pack_bias(b2_0)
    bb11, bb21 = _pack_bias(b1_1), _pack_bias(b2_1)
    wsk0 = jnp.zeros((_PF, _PF), jnp.float32).at[:_F0, :_T].set(Wskip0)
    wsk1 = jnp.eye(_PF, dtype=jnp.float32)

    cgrid = (_B, _N // _BR)

    def _full_sig(shape):
        return pl.BlockSpec((1,) + shape, lambda b, r: (b, 0, 0))

    def _row_blk(cols):
        return pl.BlockSpec((1, _BR, cols), lambda b, r: (b, r, 0))

    def _shared2(shape):
        return pl.BlockSpec(shape, lambda b, r: (0,) * len(shape))

    def conv_first(sig, w, b):
        return pl.pallas_call(
            _conv_first_kernel,
            grid=cgrid,
            in_specs=[_full_sig((_N, _PF)),
                      _row_blk(_K),
                      _row_blk(_NT * 3),
                      _row_blk(_NT * 3),
                      _shared2((kdim, _NA * _PF)),
                      _shared2((1, _NA * _PF))],
            out_specs=_row_blk(_PF),
            out_shape=jax.ShapeDtypeStruct((_B, _N, _PF), jnp.float32),
            scratch_shapes=[pltpu.VMEM((_BR, _K * _PF), jnp.float32)],
        )(sig, nidx, i3f, bcwf, w, b)

    def conv_skip(sig, w, b, skip, wskip):
        return pl.pallas_call(
            _conv_skip_kernel,
            grid=cgrid,
            in_specs=[_full_sig((_N, _PF)),
                      _row_blk(_K),
                      _row_blk(_NT * 3),
                      _row_blk(_NT * 3),
                      _shared2((kdim, _NA * _PF)),
                      _shared2((1, _NA * _PF)),
                      _row_blk(_PF),
                      _shared2((_PF, _PF))],
            out_specs=_row_blk(_PF),
            out_shape=jax.ShapeDtypeStruct((_B, _N, _PF), jnp.float32),
            scratch_shapes=[pltpu.VMEM((_BR, _K * _PF), jnp.float32)],
        )(sig, nidx, i3f, bcwf, w, b, skip, wskip)

    h = conv_first(desc, w10, bb10)
    s1 = conv_skip(h, w20, bb20, desc, wsk0)
    h = conv_first(s1, w11, bb11)
    s2 = conv_skip(h, w21, bb21, s1, wsk1)

    wp3 = jnp.zeros((_PF, _PF, _NCLS), jnp.float32
                    ).at[:_T, :_T, :].set(Wclf.reshape(_T, _T, _NCLS))
    out = pl.pallas_call(
        _head_kernel,
        grid=grid,
        in_specs=[_per_batch_spec((_N, _PF)),
                  _shared_spec((_PF, _PF, _NCLS)),
                  _shared_spec((1, _NCLS))],
        out_specs=_per_batch_spec((1, _NCLS)),
        out_shape=jax.ShapeDtypeStruct((_B, 1, _NCLS), jnp.float32),
    )(s2, wp3, bclf[None, :].astype(jnp.float32))
    return out.reshape(_B, _NCLS)
